# baseline (speedup 1.0000x reference)
import functools

import jax
import jax.numpy as jnp
from jax import lax
from jax.experimental import pallas as pl
from jax.experimental.pallas import tpu as pltpu
from jax.experimental.pallas import tpu_sc as plsc

_CHUNK = 128


@functools.cache
def _make_gather(num_rows: int, lanes: int, planes: int, batch: int):
    info = plsc.get_sparse_core_info()
    nw = info.num_cores * info.num_subcores
    nc = info.num_cores
    nl = info.num_lanes
    assert batch == nw * _CHUNK and _CHUNK % nl == 0
    plane_stride = num_rows // planes
    mesh = plsc.VectorSubcoreMesh(core_axis_name="c", subcore_axis_name="s")

    @functools.partial(
        pl.kernel,
        mesh=mesh,
        out_type=jax.ShapeDtypeStruct((planes, nw, _CHUNK, lanes), jnp.float32),
        scratch_types=[
            pltpu.VMEM((_CHUNK,), jnp.int32),
            pltpu.VMEM((planes, _CHUNK), jnp.int32),
            pltpu.VMEM((planes, _CHUNK, lanes), jnp.float32),
            pltpu.SemaphoreType.DMA,
        ],
    )
    def gather_kernel(table_hbm, idx_hbm, out_hbm, raw_v, idx_v, rows_v, sem):
        wid = lax.axis_index("s") * nc + lax.axis_index("c")
        pltpu.sync_copy(idx_hbm.at[pl.ds(wid * _CHUNK, _CHUNK)], raw_v)
        for r in range(planes):
            for j in range(_CHUNK // nl):
                sl = pl.ds(j * nl, nl)
                idx_v[r, sl] = raw_v[sl] + (r * plane_stride)
        copies = [
            pltpu.async_copy(table_hbm.at[idx_v.at[r]], rows_v.at[r], sem)
            for r in range(planes)
        ]
        for r in range(planes):
            copies[r].wait()
            pltpu.sync_copy(rows_v.at[r], out_hbm.at[r].at[wid])

    return gather_kernel


def kernel(class_number, embedding):
    num_classes, pts, ch = embedding.shape
    batch = class_number.shape[0]
    table = jnp.moveaxis(embedding, 2, 0).reshape(num_classes * ch, pts)
    idx = class_number.astype(jnp.int32)
    out = _make_gather(num_classes * ch, pts, ch, batch)(table, idx)
    return jnp.moveaxis(out.reshape(ch, batch, pts), 0, 2)
